# baseline (speedup 1.0000x reference)
import jax
import jax.numpy as jnp
from jax.experimental import pallas as pl
from jax.experimental.pallas import tpu as pltpu

TOKEN_BLOCK = 512
NBUF = 4


def _gate_kernel(x_hbm, wt_ref, o_ref, buf, sem):
    nblocks = x_hbm.shape[0] // TOKEN_BLOCK

    def copy_in(block, slot):
        pltpu.make_async_copy(
            x_hbm.at[pl.ds(block * TOKEN_BLOCK, TOKEN_BLOCK), :],
            buf.at[slot],
            sem.at[slot],
        ).start()

    for s in range(NBUF):
        copy_in(s, s)

    def body(i, _):
        slot = jax.lax.rem(i, NBUF)
        pltpu.make_async_copy(
            x_hbm.at[pl.ds(i * TOKEN_BLOCK, TOKEN_BLOCK), :],
            buf.at[slot],
            sem.at[slot],
        ).wait()
        logits = jax.lax.dot_general(
            buf[slot],
            wt_ref[...],
            dimension_numbers=(((1,), (0,)), ((), ())),
            preferred_element_type=jnp.float32,
            precision=jax.lax.Precision.DEFAULT,
        )
        o_ref[pl.ds(i * TOKEN_BLOCK, TOKEN_BLOCK), :] = (logits > 0.0).astype(
            jnp.int32
        )
        nxt = i + NBUF

        @pl.when(nxt < nblocks)
        def _():
            copy_in(nxt, slot)

        return 0

    jax.lax.fori_loop(0, nblocks, body, 0)


@jax.jit
def kernel(cls_hidden_states, gate_w):
    tokens, hidden = cls_hidden_states.shape
    num_experts = gate_w.shape[0]
    wt = gate_w.T

    return pl.pallas_call(
        _gate_kernel,
        in_specs=[
            pl.BlockSpec(memory_space=pltpu.MemorySpace.HBM),
            pl.BlockSpec(memory_space=pltpu.MemorySpace.VMEM),
        ],
        out_specs=pl.BlockSpec(memory_space=pltpu.MemorySpace.VMEM),
        out_shape=jax.ShapeDtypeStruct((tokens, num_experts), jnp.int32),
        scratch_shapes=[
            pltpu.VMEM((NBUF, TOKEN_BLOCK, hidden), jnp.float32),
            pltpu.SemaphoreType.DMA((NBUF,)),
        ],
    )(cls_hidden_states, wt)


# speedup vs baseline: 1.0215x; 1.0215x over previous
import jax
import jax.numpy as jnp
from jax.experimental import pallas as pl
from jax.experimental.pallas import tpu as pltpu

TOKEN_BLOCK = 1024


def _gate_block(x_ref, wt_ref, o_ref):
    logits = jax.lax.dot_general(
        x_ref[...],
        wt_ref[...],
        dimension_numbers=(((1,), (0,)), ((), ())),
        preferred_element_type=jnp.float32,
        precision=jax.lax.Precision.DEFAULT,
    )
    o_ref[...] = (logits > 0.0).astype(jnp.int32)


@jax.jit
def kernel(cls_hidden_states, gate_w):
    tokens, hidden = cls_hidden_states.shape
    num_experts = gate_w.shape[0]
    wt = gate_w.T

    grid = (tokens // TOKEN_BLOCK,)
    return pl.pallas_call(
        _gate_block,
        grid=grid,
        in_specs=[
            pl.BlockSpec((TOKEN_BLOCK, hidden), lambda i: (i, 0)),
            pl.BlockSpec((hidden, num_experts), lambda i: (0, 0)),
        ],
        out_specs=pl.BlockSpec((TOKEN_BLOCK, num_experts), lambda i: (i, 0)),
        out_shape=jax.ShapeDtypeStruct((tokens, num_experts), jnp.int32),
        compiler_params=pltpu.CompilerParams(
            dimension_semantics=("parallel",),
        ),
    )(cls_hidden_states, wt)


# speedup vs baseline: 1.0221x; 1.0006x over previous
import jax
import jax.numpy as jnp
from jax.experimental import pallas as pl
from jax.experimental.pallas import tpu as pltpu

BLOCK = 1024
NSUB = 8
SUB = BLOCK // NSUB


def _gate_kernel(x_hbm, wt_ref, o_ref, buf, sem):
    nblocks = x_hbm.shape[0] // BLOCK

    def sub_copy(block, slot, j):
        return pltpu.make_async_copy(
            x_hbm.at[pl.ds(block * BLOCK + j * SUB, SUB), :],
            buf.at[slot, pl.ds(j * SUB, SUB), :],
            sem.at[slot],
        )

    def issue(block, slot):
        for j in range(NSUB):
            sub_copy(block, slot, j).start()

    issue(0, 0)
    issue(1, 1)

    def body(i, _):
        slot = jax.lax.rem(i, 2)
        for j in range(NSUB):
            sub_copy(i, slot, j).wait()
        logits = jax.lax.dot_general(
            buf[slot],
            wt_ref[...],
            dimension_numbers=(((1,), (0,)), ((), ())),
            preferred_element_type=jnp.float32,
            precision=jax.lax.Precision.DEFAULT,
        )
        o_ref[pl.ds(i * BLOCK, BLOCK), :] = (logits > 0.0).astype(jnp.int32)

        @pl.when(i + 2 < nblocks)
        def _():
            issue(i + 2, slot)

        return 0

    jax.lax.fori_loop(0, nblocks, body, 0)


@jax.jit
def kernel(cls_hidden_states, gate_w):
    tokens, hidden = cls_hidden_states.shape
    num_experts = gate_w.shape[0]
    wt = gate_w.T

    return pl.pallas_call(
        _gate_kernel,
        in_specs=[
            pl.BlockSpec(memory_space=pltpu.MemorySpace.HBM),
            pl.BlockSpec(memory_space=pltpu.MemorySpace.VMEM),
        ],
        out_specs=pl.BlockSpec(memory_space=pltpu.MemorySpace.VMEM),
        out_shape=jax.ShapeDtypeStruct((tokens, num_experts), jnp.int32),
        scratch_shapes=[
            pltpu.VMEM((2, BLOCK, hidden), jnp.float32),
            pltpu.SemaphoreType.DMA((2,)),
        ],
    )(cls_hidden_states, wt)


# speedup vs baseline: 1.0930x; 1.0693x over previous
import jax
import jax.numpy as jnp
from jax.experimental import pallas as pl

TOKEN_BLOCK = 1024
GROUPS = 8
GROUP_ROWS = TOKEN_BLOCK // GROUPS


def _gate_block(x_ref, wt_ref, o_ref):
    logits = jax.lax.dot_general(
        x_ref[...],
        wt_ref[...],
        dimension_numbers=(((1,), (0,)), ((), ())),
        preferred_element_type=jnp.float32,
        precision=jax.lax.Precision.DEFAULT,
    )
    mask = (logits > 0.0).astype(jnp.int32)
    o_ref[...] = jnp.concatenate(
        [mask[j * GROUP_ROWS : (j + 1) * GROUP_ROWS, :] for j in range(GROUPS)],
        axis=1,
    )


@jax.jit
def kernel(cls_hidden_states, gate_w):
    tokens, hidden = cls_hidden_states.shape
    num_experts = gate_w.shape[0]
    wt = gate_w.T

    nblocks = tokens // TOKEN_BLOCK
    packed = pl.pallas_call(
        _gate_block,
        grid=(nblocks,),
        in_specs=[
            pl.BlockSpec((TOKEN_BLOCK, hidden), lambda i: (i, 0)),
            pl.BlockSpec((hidden, num_experts), lambda i: (0, 0)),
        ],
        out_specs=pl.BlockSpec(
            (GROUP_ROWS, GROUPS * num_experts), lambda i: (i, 0)
        ),
        out_shape=jax.ShapeDtypeStruct(
            (nblocks * GROUP_ROWS, GROUPS * num_experts), jnp.int32
        ),
    )(cls_hidden_states, wt)
    unpacked = packed.reshape(nblocks, GROUP_ROWS, GROUPS, num_experts)
    return jnp.transpose(unpacked, (0, 2, 1, 3)).reshape(tokens, num_experts)


# speedup vs baseline: 1.3077x; 1.1964x over previous
import jax
import jax.numpy as jnp
from jax.experimental import pallas as pl

TOKEN_BLOCK = 1024


def _gate_block(w_ref, x_ref, o_ref):
    logits_t = jax.lax.dot_general(
        w_ref[...],
        x_ref[...],
        dimension_numbers=(((1,), (1,)), ((), ())),
        preferred_element_type=jnp.float32,
        precision=jax.lax.Precision.DEFAULT,
    )
    o_ref[...] = (logits_t > 0.0).astype(jnp.int32)


@jax.jit
def kernel(cls_hidden_states, gate_w):
    tokens, hidden = cls_hidden_states.shape
    num_experts = gate_w.shape[0]

    grid = (tokens // TOKEN_BLOCK,)
    mask_t = pl.pallas_call(
        _gate_block,
        grid=grid,
        in_specs=[
            pl.BlockSpec((num_experts, hidden), lambda i: (0, 0)),
            pl.BlockSpec((TOKEN_BLOCK, hidden), lambda i: (i, 0)),
        ],
        out_specs=pl.BlockSpec((num_experts, TOKEN_BLOCK), lambda i: (0, i)),
        out_shape=jax.ShapeDtypeStruct((num_experts, tokens), jnp.int32),
    )(gate_w, cls_hidden_states)
    return mask_t.T
